# baseline (speedup 1.0000x reference)
import functools

import jax
import jax.numpy as jnp
from jax import lax
from jax.experimental import pallas as pl
from jax.experimental.pallas import tpu as pltpu
from jax.experimental.pallas import tpu_sc as plsc

B = 4096
L_SEQ = 200
D = 64
HALF = L_SEQ // 2

NC = 2
NS = 16
NW = NC * NS
RPW = B // NW


N_FULL_SLABS = 7812
TAIL_C0 = 1000000 - 128


def _sc_format(emb_t):
    mesh = plsc.VectorSubcoreMesh(
        core_axis_name="c", subcore_axis_name="s", num_cores=NC, num_subcores=NS
    )

    @functools.partial(
        pl.kernel,
        mesh=mesh,
        compiler_params=pltpu.CompilerParams(
            use_tc_tiling_on_sc=True, needs_layout_passes=False
        ),
        out_type=jax.ShapeDtypeStruct((500000, 128), jnp.float32),
        scratch_types=[
            pltpu.VMEM((2, 64, 128), jnp.float32),
            pltpu.VMEM((2, 64, 128), jnp.float32),
            pltpu.SemaphoreType.DMA,
            pltpu.SemaphoreType.DMA,
            pltpu.SemaphoreType.DMA,
            pltpu.SemaphoreType.DMA,
        ],
    )
    def k(src_hbm, out_hbm, cm, bf, sin0, sin1, sout0, sout1):
        wid = lax.axis_index("s") * NC + lax.axis_index("c")
        n = jnp.where(wid < 4, 245, 244)
        s0 = 244 * wid + jnp.minimum(wid, 4)
        iota = jnp.arange(16, dtype=jnp.int32)

        def in_desc(j, buf):
            sem = sin0 if buf == 0 else sin1
            off = pl.multiple_of((s0 + j) * 128, 128)
            return pltpu.make_async_copy(
                src_hbm.at[:, pl.ds(off, 128)], cm.at[buf], sem
            )

        def out_desc(j, buf):
            sem = sout0 if buf == 0 else sout1
            off = pl.multiple_of((s0 + j) * 64, 64)
            return pltpu.make_async_copy(
                bf.at[buf], out_hbm.at[pl.ds(off, 64), :], sem
            )

        def transpose(buf):
            def chunk(kb, carry):
                for kk in range(16):
                    vcol = kb * 16 + kk
                    i = kb * 8 + kk // 2
                    off = (kk % 2) * 64
                    colv = jnp.full((16,), 0, jnp.int32) + vcol
                    for g in range(4):
                        vals = plsc.load_gather(
                            cm.at[buf], [iota + g * 16, colv]
                        )
                        bf[buf, i, pl.ds(off + g * 16, 16)] = vals
                return carry

            lax.fori_loop(0, 8, chunk, 0)

        in_desc(0, 0).start()
        in_desc(1, 1).start()

        def pair(p, carry):
            for buf in range(2):
                j = 2 * p + buf
                in_desc(j, buf).wait()

                @pl.when(p > 0)
                def _():
                    out_desc(j, buf).wait()

                transpose(buf)
                out_desc(j, buf).start()

                @pl.when(j + 2 < n)
                def _():
                    in_desc(j + 2, buf).start()

            return carry

        lax.fori_loop(0, 122, pair, 0)

        @pl.when(wid < 4)
        def _():
            in_desc(244, 0).wait()
            out_desc(244, 0).wait()
            transpose(0)
            out_desc(244, 0).start()

        out_desc(0, 0).wait()
        out_desc(0, 1).wait()

    return k(emb_t)


def _tail_patch_body(dst_ref, emb_ref, out_ref):
    x = emb_ref[...]
    rows = lax.broadcasted_iota(jnp.int32, (32, 128), 0)
    cols = lax.broadcasted_iota(jnp.int32, (32, 128), 1)
    qe = (cols == 2 * rows).astype(jnp.float32)
    qo = (cols == 2 * rows + 1).astype(jnp.float32)
    contract = (((1,), (1,)), ((), ()))
    a = lax.dot_general(qe, x, contract, preferred_element_type=jnp.float32)
    b = lax.dot_general(qo, x, contract, preferred_element_type=jnp.float32)
    out_ref[...] = jnp.concatenate([a[:, :64], b[:, :64]], axis=1)


def _tc_tail_patch(k1_out, emb_t):
    return pl.pallas_call(
        _tail_patch_body,
        grid=(1,),
        in_specs=[
            pl.BlockSpec((32, 128), lambda i: (15624, 0)),
            pl.BlockSpec((64, 128), lambda i: (0, 7812)),
        ],
        out_specs=pl.BlockSpec((32, 128), lambda i: (15624, 0)),
        out_shape=jax.ShapeDtypeStruct((500000, 128), jnp.float32),
        input_output_aliases={0: 0},
    )(k1_out, emb_t)


def _sc_pool_sums(emb_table, ids2):
    mesh = plsc.VectorSubcoreMesh(
        core_axis_name="c", subcore_axis_name="s", num_cores=NC, num_subcores=NS
    )

    @functools.partial(
        pl.kernel,
        mesh=mesh,
        compiler_params=pltpu.CompilerParams(use_tc_tiling_on_sc=False),
        out_type=jax.ShapeDtypeStruct((B, D), jnp.float32),
        scratch_types=[
            pltpu.VMEM((2 * RPW, HALF), jnp.int32),
            pltpu.VMEM((2, L_SEQ, D), jnp.float32),
            pltpu.VMEM((RPW, D), jnp.float32),
            pltpu.SemaphoreType.DMA,
            pltpu.SemaphoreType.DMA,
        ],
    )
    def k(emb_hbm, ids_hbm, out_hbm, ids_v, rows_v, out_v, sem0, sem1):
        wid = lax.axis_index("s") * NC + lax.axis_index("c")
        base2 = wid * (2 * RPW)
        pltpu.sync_copy(ids_hbm.at[pl.ds(base2, 2 * RPW)], ids_v)

        def descs(row, buf):
            sem = sem0 if buf == 0 else sem1
            d0 = pltpu.make_async_copy(
                emb_hbm.at[ids_v.at[2 * row]],
                rows_v.at[buf, pl.ds(0, HALF)],
                sem,
            )
            d1 = pltpu.make_async_copy(
                emb_hbm.at[ids_v.at[2 * row + 1]],
                rows_v.at[buf, pl.ds(HALF, HALF)],
                sem,
            )
            return d0, d1

        def start(row, buf):
            d0, d1 = descs(row, buf)
            d0.start()
            d1.start()

        def wait(row, buf):
            d0, d1 = descs(row, buf)
            d0.wait()
            d1.wait()

        def accum(row, buf):
            zero = jnp.zeros((16,), jnp.float32)

            def body(l, accs):
                return tuple(
                    accs[g] + rows_v[buf, l, pl.ds(g * 16, 16)] for g in range(4)
                )

            accs = lax.fori_loop(0, L_SEQ, body, (zero,) * 4)
            for g in range(4):
                out_v[row, pl.ds(g * 16, 16)] = accs[g]

        start(0, 0)
        start(1, 1)

        def pair(p, carry):
            i0 = 2 * p
            wait(i0, 0)
            accum(i0, 0)
            start(i0 + 2, 0)
            wait(i0 + 1, 1)
            accum(i0 + 1, 1)
            start(i0 + 3, 1)
            return carry

        lax.fori_loop(0, RPW // 2 - 1, pair, 0)
        wait(RPW - 2, 0)
        accum(RPW - 2, 0)
        wait(RPW - 1, 1)
        accum(RPW - 1, 1)
        pltpu.sync_copy(out_v, out_hbm.at[pl.ds(wid * RPW, RPW)])

    return k(emb_table, ids2)


BL = 512


def _video_body(vid_ref, ids_ref, wb_ref, vb_ref, v_ref, den_ref):
    v_ref[...] = (
        jnp.dot(wb_ref[...], vid_ref[...], preferred_element_type=jnp.float32)
        + vb_ref[...]
    )
    cnt = jnp.sum((ids_ref[...] != 0).astype(jnp.float32), axis=0, keepdims=True)
    den_ref[...] = jnp.maximum(cnt, 1.0)


def _tc_video(vid2, ids_t, w_big, vid_b2):
    grid = (B // BL,)
    return pl.pallas_call(
        _video_body,
        grid=grid,
        in_specs=[
            pl.BlockSpec((12 * 256, BL), lambda i: (0, i)),
            pl.BlockSpec((L_SEQ, BL), lambda i: (0, i)),
            pl.BlockSpec((D, 12 * 256), lambda i: (0, 0)),
            pl.BlockSpec((D, 1), lambda i: (0, 0)),
        ],
        out_specs=[
            pl.BlockSpec((D, BL), lambda i: (0, i)),
            pl.BlockSpec((1, BL), lambda i: (0, i)),
        ],
        out_shape=[
            jax.ShapeDtypeStruct((D, B), jnp.float32),
            jax.ShapeDtypeStruct((1, B), jnp.float32),
        ],
    )(vid2, ids_t, w_big, vid_b2)


def _final_body(pooled_ref, v_ref, den_ref, wt_ref, tb_ref, out_ref):
    x = pooled_ref[...] / den_ref[...]
    x = (
        jnp.dot(wt_ref[...], x, preferred_element_type=jnp.float32)
        + tb_ref[...]
    )
    v = v_ref[...]
    vn = v / jnp.maximum(
        jnp.sqrt(jnp.sum(v * v, axis=0, keepdims=True)), 1e-12
    )
    xn = x / jnp.maximum(
        jnp.sqrt(jnp.sum(x * x, axis=0, keepdims=True)), 1e-12
    )
    cos = jnp.sum(vn * xn, axis=0, keepdims=True) / jnp.maximum(
        jnp.sqrt(jnp.sum(vn * vn, axis=0, keepdims=True))
        * jnp.sqrt(jnp.sum(xn * xn, axis=0, keepdims=True)),
        1e-8,
    )
    loss = jnp.sum(1.0 - cos) * (1.0 / B)
    out_ref[...] = jnp.reshape(loss, (1, 1))


def _tc_final(pooled_t, v_t, den_t, txt_w, txt_b2):
    out = pl.pallas_call(
        _final_body,
        grid=(1,),
        in_specs=[
            pl.BlockSpec((D, B), lambda i: (0, 0)),
            pl.BlockSpec((D, B), lambda i: (0, 0)),
            pl.BlockSpec((1, B), lambda i: (0, 0)),
            pl.BlockSpec((D, D), lambda i: (0, 0)),
            pl.BlockSpec((D, 1), lambda i: (0, 0)),
        ],
        out_specs=pl.BlockSpec((1, 1), lambda i: (0, 0)),
        out_shape=jax.ShapeDtypeStruct((1, 1), jnp.float32),
    )(pooled_t, v_t, den_t, txt_w, txt_b2)
    return out[0, 0]


def kernel(video, caption_ids, emb_table, txt_w, txt_b, vid_w, vid_b):
    ids = caption_ids.astype(jnp.int32)
    emb_t = emb_table.T
    table_rm = _tc_tail_patch(_sc_format(emb_t), emb_t).reshape(1000000, D)
    pooled = _sc_pool_sums(table_rm, ids.reshape(B * 2, HALF))
    vid2 = video.transpose(1, 2, 3, 4, 0).reshape(12 * 256, B)
    ids_t = ids.T
    w_big = jnp.tile(jnp.repeat(vid_w * (1.0 / 1024.0), 256, axis=1), (1, 4))
    v_t, den_t = _tc_video(vid2, ids_t, w_big, vid_b.reshape(D, 1))
    return _tc_final(pooled.T, v_t, den_t, txt_w, txt_b.reshape(D, 1))


# speedup vs baseline: 1.1899x; 1.1899x over previous
import functools

import jax
import jax.numpy as jnp
from jax import lax
from jax.experimental import pallas as pl
from jax.experimental.pallas import tpu as pltpu
from jax.experimental.pallas import tpu_sc as plsc

B = 4096
L_SEQ = 200
D = 64
HALF = L_SEQ // 2

NC = 2
NS = 16
NW = NC * NS
RPW = B // NW


N_FULL_SLABS = 7812
TAIL_C0 = 1000000 - 128


def _sc_format(emb_t):
    mesh = plsc.VectorSubcoreMesh(
        core_axis_name="c", subcore_axis_name="s", num_cores=NC, num_subcores=NS
    )

    @functools.partial(
        pl.kernel,
        mesh=mesh,
        compiler_params=pltpu.CompilerParams(
            use_tc_tiling_on_sc=True, needs_layout_passes=False
        ),
        out_type=jax.ShapeDtypeStruct((500000, 128), jnp.float32),
        scratch_types=[
            pltpu.VMEM((2, 64, 128), jnp.float32),
            pltpu.VMEM((2, 64, 128), jnp.float32),
            pltpu.SemaphoreType.DMA,
            pltpu.SemaphoreType.DMA,
            pltpu.SemaphoreType.DMA,
            pltpu.SemaphoreType.DMA,
        ],
    )
    def k(src_hbm, out_hbm, cm, bf, sin0, sin1, sout0, sout1):
        wid = lax.axis_index("s") * NC + lax.axis_index("c")
        n = jnp.where(wid < 4, 245, 244)
        s0 = 244 * wid + jnp.minimum(wid, 4)
        iota = jnp.arange(16, dtype=jnp.int32)

        def in_desc(j, buf):
            sem = sin0 if buf == 0 else sin1
            off = pl.multiple_of((s0 + j) * 128, 128)
            return pltpu.make_async_copy(
                src_hbm.at[:, pl.ds(off, 128)], cm.at[buf], sem
            )

        def out_desc(j, buf):
            sem = sout0 if buf == 0 else sout1
            off = pl.multiple_of((s0 + j) * 64, 64)
            return pltpu.make_async_copy(
                bf.at[buf], out_hbm.at[pl.ds(off, 64), :], sem
            )

        base_parity = (iota % 2) * 64
        half = iota // 2

        def transpose(buf):
            for d in range(64):
                colv = base_parity + d
                for c in range(8):
                    vals = cm[buf, d, pl.ds(16 * c, 16)]
                    rowv = half + 8 * c
                    plsc.store_scatter(bf.at[buf], [rowv, colv], vals)

        in_desc(0, 0).start()
        in_desc(1, 1).start()

        def pair(p, carry):
            for buf in range(2):
                j = 2 * p + buf
                in_desc(j, buf).wait()

                @pl.when(p > 0)
                def _():
                    out_desc(j, buf).wait()

                transpose(buf)
                out_desc(j, buf).start()

                @pl.when(j + 2 < n)
                def _():
                    in_desc(j + 2, buf).start()

            return carry

        lax.fori_loop(0, 122, pair, 0)

        @pl.when(wid < 4)
        def _():
            in_desc(244, 0).wait()
            out_desc(244, 0).wait()
            transpose(0)
            out_desc(244, 0).start()

        out_desc(0, 0).wait()
        out_desc(0, 1).wait()

    return k(emb_t)


def _tail_patch_body(dst_ref, emb_ref, out_ref):
    x = emb_ref[...]
    rows = lax.broadcasted_iota(jnp.int32, (32, 128), 0)
    cols = lax.broadcasted_iota(jnp.int32, (32, 128), 1)
    qe = (cols == 2 * rows).astype(jnp.float32)
    qo = (cols == 2 * rows + 1).astype(jnp.float32)
    contract = (((1,), (1,)), ((), ()))
    a = lax.dot_general(qe, x, contract, preferred_element_type=jnp.float32)
    b = lax.dot_general(qo, x, contract, preferred_element_type=jnp.float32)
    out_ref[...] = jnp.concatenate([a[:, :64], b[:, :64]], axis=1)


def _tc_tail_patch(k1_out, emb_t):
    return pl.pallas_call(
        _tail_patch_body,
        grid=(1,),
        in_specs=[
            pl.BlockSpec((32, 128), lambda i: (15624, 0)),
            pl.BlockSpec((64, 128), lambda i: (0, 7812)),
        ],
        out_specs=pl.BlockSpec((32, 128), lambda i: (15624, 0)),
        out_shape=jax.ShapeDtypeStruct((500000, 128), jnp.float32),
        input_output_aliases={0: 0},
    )(k1_out, emb_t)


def _sc_pool_sums(emb_table, ids2):
    mesh = plsc.VectorSubcoreMesh(
        core_axis_name="c", subcore_axis_name="s", num_cores=NC, num_subcores=NS
    )

    @functools.partial(
        pl.kernel,
        mesh=mesh,
        compiler_params=pltpu.CompilerParams(use_tc_tiling_on_sc=False),
        out_type=jax.ShapeDtypeStruct((B, D), jnp.float32),
        scratch_types=[
            pltpu.VMEM((2 * RPW, HALF), jnp.int32),
            pltpu.VMEM((2, L_SEQ, D), jnp.float32),
            pltpu.VMEM((RPW, D), jnp.float32),
            pltpu.SemaphoreType.DMA,
            pltpu.SemaphoreType.DMA,
        ],
    )
    def k(emb_hbm, ids_hbm, out_hbm, ids_v, rows_v, out_v, sem0, sem1):
        wid = lax.axis_index("s") * NC + lax.axis_index("c")
        base2 = wid * (2 * RPW)
        pltpu.sync_copy(ids_hbm.at[pl.ds(base2, 2 * RPW)], ids_v)

        def descs(row, buf):
            sem = sem0 if buf == 0 else sem1
            d0 = pltpu.make_async_copy(
                emb_hbm.at[ids_v.at[2 * row]],
                rows_v.at[buf, pl.ds(0, HALF)],
                sem,
            )
            d1 = pltpu.make_async_copy(
                emb_hbm.at[ids_v.at[2 * row + 1]],
                rows_v.at[buf, pl.ds(HALF, HALF)],
                sem,
            )
            return d0, d1

        def start(row, buf):
            d0, d1 = descs(row, buf)
            d0.start()
            d1.start()

        def wait(row, buf):
            d0, d1 = descs(row, buf)
            d0.wait()
            d1.wait()

        def accum(row, buf):
            zero = jnp.zeros((16,), jnp.float32)

            def body(l, accs):
                return tuple(
                    accs[g] + rows_v[buf, l, pl.ds(g * 16, 16)] for g in range(4)
                )

            accs = lax.fori_loop(0, L_SEQ, body, (zero,) * 4)
            for g in range(4):
                out_v[row, pl.ds(g * 16, 16)] = accs[g]

        start(0, 0)
        start(1, 1)

        def pair(p, carry):
            i0 = 2 * p
            wait(i0, 0)
            accum(i0, 0)
            start(i0 + 2, 0)
            wait(i0 + 1, 1)
            accum(i0 + 1, 1)
            start(i0 + 3, 1)
            return carry

        lax.fori_loop(0, RPW // 2 - 1, pair, 0)
        wait(RPW - 2, 0)
        accum(RPW - 2, 0)
        wait(RPW - 1, 1)
        accum(RPW - 1, 1)
        pltpu.sync_copy(out_v, out_hbm.at[pl.ds(wid * RPW, RPW)])

    return k(emb_table, ids2)


BL = 512


def _video_body(vid_ref, ids_ref, wb_ref, vb_ref, v_ref, den_ref):
    v_ref[...] = (
        jnp.dot(wb_ref[...], vid_ref[...], preferred_element_type=jnp.float32)
        + vb_ref[...]
    )
    cnt = jnp.sum((ids_ref[...] != 0).astype(jnp.float32), axis=0, keepdims=True)
    den_ref[...] = jnp.maximum(cnt, 1.0)


def _tc_video(vid2, ids_t, w_big, vid_b2):
    grid = (B // BL,)
    return pl.pallas_call(
        _video_body,
        grid=grid,
        in_specs=[
            pl.BlockSpec((12 * 256, BL), lambda i: (0, i)),
            pl.BlockSpec((L_SEQ, BL), lambda i: (0, i)),
            pl.BlockSpec((D, 12 * 256), lambda i: (0, 0)),
            pl.BlockSpec((D, 1), lambda i: (0, 0)),
        ],
        out_specs=[
            pl.BlockSpec((D, BL), lambda i: (0, i)),
            pl.BlockSpec((1, BL), lambda i: (0, i)),
        ],
        out_shape=[
            jax.ShapeDtypeStruct((D, B), jnp.float32),
            jax.ShapeDtypeStruct((1, B), jnp.float32),
        ],
    )(vid2, ids_t, w_big, vid_b2)


def _final_body(pooled_ref, v_ref, den_ref, wt_ref, tb_ref, out_ref):
    x = pooled_ref[...] / den_ref[...]
    x = (
        jnp.dot(wt_ref[...], x, preferred_element_type=jnp.float32)
        + tb_ref[...]
    )
    v = v_ref[...]
    vn = v / jnp.maximum(
        jnp.sqrt(jnp.sum(v * v, axis=0, keepdims=True)), 1e-12
    )
    xn = x / jnp.maximum(
        jnp.sqrt(jnp.sum(x * x, axis=0, keepdims=True)), 1e-12
    )
    cos = jnp.sum(vn * xn, axis=0, keepdims=True) / jnp.maximum(
        jnp.sqrt(jnp.sum(vn * vn, axis=0, keepdims=True))
        * jnp.sqrt(jnp.sum(xn * xn, axis=0, keepdims=True)),
        1e-8,
    )
    loss = jnp.sum(1.0 - cos) * (1.0 / B)
    out_ref[...] = jnp.reshape(loss, (1, 1))


def _tc_final(pooled_t, v_t, den_t, txt_w, txt_b2):
    out = pl.pallas_call(
        _final_body,
        grid=(1,),
        in_specs=[
            pl.BlockSpec((D, B), lambda i: (0, 0)),
            pl.BlockSpec((D, B), lambda i: (0, 0)),
            pl.BlockSpec((1, B), lambda i: (0, 0)),
            pl.BlockSpec((D, D), lambda i: (0, 0)),
            pl.BlockSpec((D, 1), lambda i: (0, 0)),
        ],
        out_specs=pl.BlockSpec((1, 1), lambda i: (0, 0)),
        out_shape=jax.ShapeDtypeStruct((1, 1), jnp.float32),
    )(pooled_t, v_t, den_t, txt_w, txt_b2)
    return out[0, 0]


def kernel(video, caption_ids, emb_table, txt_w, txt_b, vid_w, vid_b):
    ids = caption_ids.astype(jnp.int32)
    emb_t = emb_table.T
    table_rm = _tc_tail_patch(_sc_format(emb_t), emb_t).reshape(1000000, D)
    pooled = _sc_pool_sums(table_rm, ids.reshape(B * 2, HALF))
    vid2 = video.transpose(1, 2, 3, 4, 0).reshape(12 * 256, B)
    ids_t = ids.T
    w_big = jnp.tile(jnp.repeat(vid_w * (1.0 / 1024.0), 256, axis=1), (1, 4))
    v_t, den_t = _tc_video(vid2, ids_t, w_big, vid_b.reshape(D, 1))
    return _tc_final(pooled.T, v_t, den_t, txt_w, txt_b.reshape(D, 1))


# speedup vs baseline: 1.2085x; 1.0156x over previous
import functools

import jax
import jax.numpy as jnp
from jax import lax
from jax.experimental import pallas as pl
from jax.experimental.pallas import tpu as pltpu
from jax.experimental.pallas import tpu_sc as plsc

B = 4096
L_SEQ = 200
D = 64
HALF = L_SEQ // 2

NC = 2
NS = 16
NW = NC * NS
RPW = B // NW


SLAB_W = 384
SLAB_R = SLAB_W // 2


def _sc_format(emb_t):
    mesh = plsc.VectorSubcoreMesh(
        core_axis_name="c", subcore_axis_name="s", num_cores=NC, num_subcores=NS
    )

    @functools.partial(
        pl.kernel,
        mesh=mesh,
        compiler_params=pltpu.CompilerParams(
            use_tc_tiling_on_sc=True, needs_layout_passes=False
        ),
        out_type=jax.ShapeDtypeStruct((500000, 128), jnp.float32),
        scratch_types=[
            pltpu.VMEM((2, 64, SLAB_W), jnp.float32),
            pltpu.VMEM((2, SLAB_R, 128), jnp.float32),
            pltpu.SemaphoreType.DMA,
            pltpu.SemaphoreType.DMA,
            pltpu.SemaphoreType.DMA,
            pltpu.SemaphoreType.DMA,
        ],
    )
    def k(src_hbm, out_hbm, cm, bf, sin0, sin1, sout0, sout1):
        wid = lax.axis_index("s") * NC + lax.axis_index("c")
        n = jnp.where(wid < 12, 82, 81)
        s0 = 81 * wid + jnp.minimum(wid, 12)
        iota = jnp.arange(16, dtype=jnp.int32)

        def in_desc(j, buf):
            sem = sin0 if buf == 0 else sin1
            off = pl.multiple_of((s0 + j) * SLAB_W, 128)
            return pltpu.make_async_copy(
                src_hbm.at[:, pl.ds(off, SLAB_W)], cm.at[buf], sem
            )

        def out_desc(j, buf):
            sem = sout0 if buf == 0 else sout1
            off = pl.multiple_of((s0 + j) * SLAB_R, 64)
            return pltpu.make_async_copy(
                bf.at[buf], out_hbm.at[pl.ds(off, SLAB_R), :], sem
            )

        base_parity = (iota % 2) * 64
        half = iota // 2

        def transpose(buf):
            def dbody(d, carry):
                colv = base_parity + d
                for c in range(SLAB_W // 16):
                    vals = cm[buf, d, pl.ds(16 * c, 16)]
                    rowv = half + 8 * c
                    plsc.store_scatter(bf.at[buf], [rowv, colv], vals)
                return carry

            lax.fori_loop(0, 64, dbody, 0)

        in_desc(0, 0).start()
        in_desc(1, 1).start()

        def pair(p, carry):
            for buf in range(2):
                j = 2 * p + buf
                in_desc(j, buf).wait()

                @pl.when(p > 0)
                def _():
                    out_desc(j, buf).wait()

                transpose(buf)
                out_desc(j, buf).start()

                @pl.when(j + 2 < n)
                def _():
                    in_desc(j + 2, buf).start()

            return carry

        lax.fori_loop(0, 40, pair, 0)

        in_desc(80, 0).wait()
        out_desc(80, 0).wait()
        transpose(0)
        out_desc(80, 0).start()

        @pl.when(wid < 12)
        def _():
            in_desc(81, 1).wait()
            out_desc(81, 1).wait()
            transpose(1)
            out_desc(81, 1).start()

        out_desc(0, 0).wait()
        out_desc(0, 1).wait()

    return k(emb_t)


def _tail_patch_body(dst_ref, emb_ref, out_ref):
    x = emb_ref[...]
    rows = lax.broadcasted_iota(jnp.int32, (32, 128), 0)
    cols = lax.broadcasted_iota(jnp.int32, (32, 128), 1)
    qe = (cols == 2 * rows).astype(jnp.float32)
    qo = (cols == 2 * rows + 1).astype(jnp.float32)
    contract = (((1,), (1,)), ((), ()))
    a = lax.dot_general(qe, x, contract, preferred_element_type=jnp.float32)
    b = lax.dot_general(qo, x, contract, preferred_element_type=jnp.float32)
    out_ref[...] = jnp.concatenate([a[:, :64], b[:, :64]], axis=1)


def _tc_tail_patch(k1_out, emb_t):
    return pl.pallas_call(
        _tail_patch_body,
        grid=(1,),
        in_specs=[
            pl.BlockSpec((32, 128), lambda i: (15624, 0)),
            pl.BlockSpec((64, 128), lambda i: (0, 7812)),
        ],
        out_specs=pl.BlockSpec((32, 128), lambda i: (15624, 0)),
        out_shape=jax.ShapeDtypeStruct((500000, 128), jnp.float32),
        input_output_aliases={0: 0},
    )(k1_out, emb_t)


def _sc_pool_sums(emb_table, ids2):
    mesh = plsc.VectorSubcoreMesh(
        core_axis_name="c", subcore_axis_name="s", num_cores=NC, num_subcores=NS
    )

    @functools.partial(
        pl.kernel,
        mesh=mesh,
        compiler_params=pltpu.CompilerParams(use_tc_tiling_on_sc=False),
        out_type=jax.ShapeDtypeStruct((B, D), jnp.float32),
        scratch_types=[
            pltpu.VMEM((2 * RPW, HALF), jnp.int32),
            pltpu.VMEM((2, L_SEQ, D), jnp.float32),
            pltpu.VMEM((RPW, D), jnp.float32),
            pltpu.SemaphoreType.DMA,
            pltpu.SemaphoreType.DMA,
        ],
    )
    def k(emb_hbm, ids_hbm, out_hbm, ids_v, rows_v, out_v, sem0, sem1):
        wid = lax.axis_index("s") * NC + lax.axis_index("c")
        base2 = wid * (2 * RPW)
        pltpu.sync_copy(ids_hbm.at[pl.ds(base2, 2 * RPW)], ids_v)

        def descs(row, buf):
            sem = sem0 if buf == 0 else sem1
            d0 = pltpu.make_async_copy(
                emb_hbm.at[ids_v.at[2 * row]],
                rows_v.at[buf, pl.ds(0, HALF)],
                sem,
            )
            d1 = pltpu.make_async_copy(
                emb_hbm.at[ids_v.at[2 * row + 1]],
                rows_v.at[buf, pl.ds(HALF, HALF)],
                sem,
            )
            return d0, d1

        def start(row, buf):
            d0, d1 = descs(row, buf)
            d0.start()
            d1.start()

        def wait(row, buf):
            d0, d1 = descs(row, buf)
            d0.wait()
            d1.wait()

        def accum(row, buf):
            zero = jnp.zeros((16,), jnp.float32)

            def body(l, accs):
                return tuple(
                    accs[g] + rows_v[buf, l, pl.ds(g * 16, 16)] for g in range(4)
                )

            accs = lax.fori_loop(0, L_SEQ, body, (zero,) * 4)
            for g in range(4):
                out_v[row, pl.ds(g * 16, 16)] = accs[g]

        start(0, 0)
        start(1, 1)

        def pair(p, carry):
            i0 = 2 * p
            wait(i0, 0)
            accum(i0, 0)
            start(i0 + 2, 0)
            wait(i0 + 1, 1)
            accum(i0 + 1, 1)
            start(i0 + 3, 1)
            return carry

        lax.fori_loop(0, RPW // 2 - 1, pair, 0)
        wait(RPW - 2, 0)
        accum(RPW - 2, 0)
        wait(RPW - 1, 1)
        accum(RPW - 1, 1)
        pltpu.sync_copy(out_v, out_hbm.at[pl.ds(wid * RPW, RPW)])

    return k(emb_table, ids2)


BL = 512


def _video_body(vid_ref, ids_ref, wb_ref, vb_ref, v_ref, den_ref):
    v_ref[...] = (
        jnp.dot(wb_ref[...], vid_ref[...], preferred_element_type=jnp.float32)
        + vb_ref[...]
    )
    cnt = jnp.sum((ids_ref[...] != 0).astype(jnp.float32), axis=0, keepdims=True)
    den_ref[...] = jnp.maximum(cnt, 1.0)


def _tc_video(vid2, ids_t, w_big, vid_b2):
    grid = (B // BL,)
    return pl.pallas_call(
        _video_body,
        grid=grid,
        in_specs=[
            pl.BlockSpec((12 * 256, BL), lambda i: (0, i)),
            pl.BlockSpec((L_SEQ, BL), lambda i: (0, i)),
            pl.BlockSpec((D, 12 * 256), lambda i: (0, 0)),
            pl.BlockSpec((D, 1), lambda i: (0, 0)),
        ],
        out_specs=[
            pl.BlockSpec((D, BL), lambda i: (0, i)),
            pl.BlockSpec((1, BL), lambda i: (0, i)),
        ],
        out_shape=[
            jax.ShapeDtypeStruct((D, B), jnp.float32),
            jax.ShapeDtypeStruct((1, B), jnp.float32),
        ],
    )(vid2, ids_t, w_big, vid_b2)


def _final_body(pooled_ref, v_ref, den_ref, wt_ref, tb_ref, out_ref):
    x = pooled_ref[...] / den_ref[...]
    x = (
        jnp.dot(wt_ref[...], x, preferred_element_type=jnp.float32)
        + tb_ref[...]
    )
    v = v_ref[...]
    vn = v / jnp.maximum(
        jnp.sqrt(jnp.sum(v * v, axis=0, keepdims=True)), 1e-12
    )
    xn = x / jnp.maximum(
        jnp.sqrt(jnp.sum(x * x, axis=0, keepdims=True)), 1e-12
    )
    cos = jnp.sum(vn * xn, axis=0, keepdims=True) / jnp.maximum(
        jnp.sqrt(jnp.sum(vn * vn, axis=0, keepdims=True))
        * jnp.sqrt(jnp.sum(xn * xn, axis=0, keepdims=True)),
        1e-8,
    )
    loss = jnp.sum(1.0 - cos) * (1.0 / B)
    out_ref[...] = jnp.reshape(loss, (1, 1))


def _tc_final(pooled_t, v_t, den_t, txt_w, txt_b2):
    out = pl.pallas_call(
        _final_body,
        grid=(1,),
        in_specs=[
            pl.BlockSpec((D, B), lambda i: (0, 0)),
            pl.BlockSpec((D, B), lambda i: (0, 0)),
            pl.BlockSpec((1, B), lambda i: (0, 0)),
            pl.BlockSpec((D, D), lambda i: (0, 0)),
            pl.BlockSpec((D, 1), lambda i: (0, 0)),
        ],
        out_specs=pl.BlockSpec((1, 1), lambda i: (0, 0)),
        out_shape=jax.ShapeDtypeStruct((1, 1), jnp.float32),
    )(pooled_t, v_t, den_t, txt_w, txt_b2)
    return out[0, 0]


def kernel(video, caption_ids, emb_table, txt_w, txt_b, vid_w, vid_b):
    ids = caption_ids.astype(jnp.int32)
    emb_t = emb_table.T
    table_rm = _tc_tail_patch(_sc_format(emb_t), emb_t).reshape(1000000, D)
    pooled = _sc_pool_sums(table_rm, ids.reshape(B * 2, HALF))
    vid2 = video.transpose(1, 2, 3, 4, 0).reshape(12 * 256, B)
    ids_t = ids.T
    w_big = jnp.tile(jnp.repeat(vid_w * (1.0 / 1024.0), 256, axis=1), (1, 4))
    v_t, den_t = _tc_video(vid2, ids_t, w_big, vid_b.reshape(D, 1))
    return _tc_final(pooled.T, v_t, den_t, txt_w, txt_b.reshape(D, 1))


# speedup vs baseline: 2.2377x; 1.8517x over previous
import functools

import jax
import jax.numpy as jnp
from jax import lax
from jax.experimental import pallas as pl
from jax.experimental.pallas import tpu as pltpu
from jax.experimental.pallas import tpu_sc as plsc

B = 4096
L_SEQ = 200
D = 64
HALF = L_SEQ // 2

NC = 2
NS = 16
NW = NC * NS
RPW = B // NW


def _sc_pool_sums(emb_table, ids2):
    mesh = plsc.VectorSubcoreMesh(
        core_axis_name="c", subcore_axis_name="s", num_cores=NC, num_subcores=NS
    )

    @functools.partial(
        pl.kernel,
        mesh=mesh,
        compiler_params=pltpu.CompilerParams(use_tc_tiling_on_sc=False),
        out_type=jax.ShapeDtypeStruct((B, D), jnp.float32),
        scratch_types=[
            pltpu.VMEM((2 * RPW, HALF), jnp.int32),
            pltpu.VMEM((2, L_SEQ, D), jnp.float32),
            pltpu.VMEM((RPW, D), jnp.float32),
            pltpu.SemaphoreType.DMA,
            pltpu.SemaphoreType.DMA,
        ],
    )
    def k(emb_hbm, ids_hbm, out_hbm, ids_v, rows_v, out_v, sem0, sem1):
        wid = lax.axis_index("s") * NC + lax.axis_index("c")
        base2 = wid * (2 * RPW)
        pltpu.sync_copy(ids_hbm.at[pl.ds(base2, 2 * RPW)], ids_v)

        def descs(row, buf):
            sem = sem0 if buf == 0 else sem1
            d0 = pltpu.make_async_copy(
                emb_hbm.at[ids_v.at[2 * row]],
                rows_v.at[buf, pl.ds(0, HALF)],
                sem,
            )
            d1 = pltpu.make_async_copy(
                emb_hbm.at[ids_v.at[2 * row + 1]],
                rows_v.at[buf, pl.ds(HALF, HALF)],
                sem,
            )
            return d0, d1

        def start(row, buf):
            d0, d1 = descs(row, buf)
            d0.start()
            d1.start()

        def wait(row, buf):
            d0, d1 = descs(row, buf)
            d0.wait()
            d1.wait()

        def accum(row, buf):
            zero = jnp.zeros((16,), jnp.float32)

            def body(l, accs):
                a = list(accs)
                for g in range(4):
                    a[g] = a[g] + rows_v[buf, 2 * l, pl.ds(g * 16, 16)]
                for g in range(4):
                    a[4 + g] = a[4 + g] + rows_v[buf, 2 * l + 1, pl.ds(g * 16, 16)]
                return tuple(a)

            accs = lax.fori_loop(0, L_SEQ // 2, body, (zero,) * 8)
            for g in range(4):
                out_v[row, pl.ds(g * 16, 16)] = accs[g] + accs[4 + g]

        start(0, 0)
        start(1, 1)

        def pair(p, carry):
            i0 = 2 * p
            wait(i0, 0)
            accum(i0, 0)
            start(i0 + 2, 0)
            wait(i0 + 1, 1)
            accum(i0 + 1, 1)
            start(i0 + 3, 1)
            return carry

        lax.fori_loop(0, RPW // 2 - 1, pair, 0)
        wait(RPW - 2, 0)
        accum(RPW - 2, 0)
        wait(RPW - 1, 1)
        accum(RPW - 1, 1)
        pltpu.sync_copy(out_v, out_hbm.at[pl.ds(wid * RPW, RPW)])

    return k(emb_table, ids2)


BL = 512


def _video_body(vid_ref, ids_ref, wb_ref, vb_ref, v_ref, den_ref):
    v_ref[...] = (
        jnp.dot(wb_ref[...], vid_ref[...], preferred_element_type=jnp.float32)
        + vb_ref[...]
    )
    cnt = jnp.sum((ids_ref[...] != 0).astype(jnp.float32), axis=0, keepdims=True)
    den_ref[...] = jnp.maximum(cnt, 1.0)


def _tc_video(vid2, ids_t, w_big, vid_b2):
    grid = (B // BL,)
    return pl.pallas_call(
        _video_body,
        grid=grid,
        in_specs=[
            pl.BlockSpec((12 * 256, BL), lambda i: (0, i)),
            pl.BlockSpec((L_SEQ, BL), lambda i: (0, i)),
            pl.BlockSpec((D, 12 * 256), lambda i: (0, 0)),
            pl.BlockSpec((D, 1), lambda i: (0, 0)),
        ],
        out_specs=[
            pl.BlockSpec((D, BL), lambda i: (0, i)),
            pl.BlockSpec((1, BL), lambda i: (0, i)),
        ],
        out_shape=[
            jax.ShapeDtypeStruct((D, B), jnp.float32),
            jax.ShapeDtypeStruct((1, B), jnp.float32),
        ],
    )(vid2, ids_t, w_big, vid_b2)


def _final_body(pooled_ref, v_ref, den_ref, wt_ref, tb_ref, out_ref):
    x = pooled_ref[...] / den_ref[...]
    x = (
        jnp.dot(wt_ref[...], x, preferred_element_type=jnp.float32)
        + tb_ref[...]
    )
    v = v_ref[...]
    vn = v / jnp.maximum(
        jnp.sqrt(jnp.sum(v * v, axis=0, keepdims=True)), 1e-12
    )
    xn = x / jnp.maximum(
        jnp.sqrt(jnp.sum(x * x, axis=0, keepdims=True)), 1e-12
    )
    cos = jnp.sum(vn * xn, axis=0, keepdims=True) / jnp.maximum(
        jnp.sqrt(jnp.sum(vn * vn, axis=0, keepdims=True))
        * jnp.sqrt(jnp.sum(xn * xn, axis=0, keepdims=True)),
        1e-8,
    )
    loss = jnp.sum(1.0 - cos) * (1.0 / B)
    out_ref[...] = jnp.reshape(loss, (1, 1))


def _tc_final(pooled_t, v_t, den_t, txt_w, txt_b2):
    out = pl.pallas_call(
        _final_body,
        grid=(1,),
        in_specs=[
            pl.BlockSpec((D, B), lambda i: (0, 0)),
            pl.BlockSpec((D, B), lambda i: (0, 0)),
            pl.BlockSpec((1, B), lambda i: (0, 0)),
            pl.BlockSpec((D, D), lambda i: (0, 0)),
            pl.BlockSpec((D, 1), lambda i: (0, 0)),
        ],
        out_specs=pl.BlockSpec((1, 1), lambda i: (0, 0)),
        out_shape=jax.ShapeDtypeStruct((1, 1), jnp.float32),
    )(pooled_t, v_t, den_t, txt_w, txt_b2)
    return out[0, 0]


def kernel(video, caption_ids, emb_table, txt_w, txt_b, vid_w, vid_b):
    ids = caption_ids.astype(jnp.int32)
    pooled = _sc_pool_sums(emb_table, ids.reshape(B * 2, HALF))
    vid2 = video.transpose(1, 2, 3, 4, 0).reshape(12 * 256, B)
    ids_t = ids.T
    w_big = jnp.tile(jnp.repeat(vid_w * (1.0 / 1024.0), 256, axis=1), (1, 4))
    v_t, den_t = _tc_video(vid2, ids_t, w_big, vid_b.reshape(D, 1))
    return _tc_final(pooled.T, v_t, den_t, txt_w, txt_b.reshape(D, 1))


# speedup vs baseline: 2.2484x; 1.0048x over previous
import functools

import jax
import jax.numpy as jnp
from jax import lax
from jax.experimental import pallas as pl
from jax.experimental.pallas import tpu as pltpu
from jax.experimental.pallas import tpu_sc as plsc

B = 4096
L_SEQ = 200
D = 64
HALF = L_SEQ // 2

NC = 2
NS = 16
NW = NC * NS
RPW = B // NW


def _sc_pool_sums(emb_table, ids2):
    mesh = plsc.VectorSubcoreMesh(
        core_axis_name="c", subcore_axis_name="s", num_cores=NC, num_subcores=NS
    )

    @functools.partial(
        pl.kernel,
        mesh=mesh,
        compiler_params=pltpu.CompilerParams(use_tc_tiling_on_sc=False),
        out_type=jax.ShapeDtypeStruct((B, D), jnp.float32),
        scratch_types=[
            pltpu.VMEM((2 * RPW, HALF), jnp.int32),
            pltpu.VMEM((2, L_SEQ, D), jnp.float32),
            pltpu.VMEM((RPW, D), jnp.float32),
            pltpu.SemaphoreType.DMA,
            pltpu.SemaphoreType.DMA,
        ],
    )
    def k(emb_hbm, ids_hbm, out_hbm, ids_v, rows_v, out_v, sem0, sem1):
        wid = lax.axis_index("s") * NC + lax.axis_index("c")
        base2 = wid * (2 * RPW)
        pltpu.sync_copy(ids_hbm.at[pl.ds(base2, 2 * RPW)], ids_v)

        def descs(row, buf):
            sem = sem0 if buf == 0 else sem1
            d0 = pltpu.make_async_copy(
                emb_hbm.at[ids_v.at[2 * row]],
                rows_v.at[buf, pl.ds(0, HALF)],
                sem,
            )
            d1 = pltpu.make_async_copy(
                emb_hbm.at[ids_v.at[2 * row + 1]],
                rows_v.at[buf, pl.ds(HALF, HALF)],
                sem,
            )
            return d0, d1

        def start(row, buf):
            d0, d1 = descs(row, buf)
            d0.start()
            d1.start()

        def wait(row, buf):
            d0, d1 = descs(row, buf)
            d0.wait()
            d1.wait()

        def accum(row, buf):
            zero = jnp.zeros((16,), jnp.float32)

            def body(l, accs):
                a = list(accs)
                for u in range(4):
                    for g in range(4):
                        k = (u % 2) * 4 + g
                        a[k] = a[k] + rows_v[buf, 4 * l + u, pl.ds(g * 16, 16)]
                return tuple(a)

            accs = lax.fori_loop(0, L_SEQ // 4, body, (zero,) * 8)
            for g in range(4):
                out_v[row, pl.ds(g * 16, 16)] = accs[g] + accs[4 + g]

        start(0, 0)
        start(1, 1)

        def pair(p, carry):
            i0 = 2 * p
            wait(i0, 0)
            accum(i0, 0)
            start(i0 + 2, 0)
            wait(i0 + 1, 1)
            accum(i0 + 1, 1)
            start(i0 + 3, 1)
            return carry

        lax.fori_loop(0, RPW // 2 - 1, pair, 0)
        wait(RPW - 2, 0)
        accum(RPW - 2, 0)
        wait(RPW - 1, 1)
        accum(RPW - 1, 1)
        pltpu.sync_copy(out_v, out_hbm.at[pl.ds(wid * RPW, RPW)])

    return k(emb_table, ids2)


BL = 512


def _video_body(vid_ref, ids_ref, wb_ref, vb_ref, v_ref, den_ref):
    v_ref[...] = (
        jnp.dot(wb_ref[...], vid_ref[...], preferred_element_type=jnp.float32)
        + vb_ref[...]
    )
    cnt = jnp.sum((ids_ref[...] != 0).astype(jnp.float32), axis=0, keepdims=True)
    den_ref[...] = jnp.maximum(cnt, 1.0)


def _tc_video(vid2, ids_t, w_big, vid_b2):
    grid = (B // BL,)
    return pl.pallas_call(
        _video_body,
        grid=grid,
        in_specs=[
            pl.BlockSpec((12 * 256, BL), lambda i: (0, i)),
            pl.BlockSpec((L_SEQ, BL), lambda i: (0, i)),
            pl.BlockSpec((D, 12 * 256), lambda i: (0, 0)),
            pl.BlockSpec((D, 1), lambda i: (0, 0)),
        ],
        out_specs=[
            pl.BlockSpec((D, BL), lambda i: (0, i)),
            pl.BlockSpec((1, BL), lambda i: (0, i)),
        ],
        out_shape=[
            jax.ShapeDtypeStruct((D, B), jnp.float32),
            jax.ShapeDtypeStruct((1, B), jnp.float32),
        ],
    )(vid2, ids_t, w_big, vid_b2)


def _final_body(pooled_ref, v_ref, den_ref, wt_ref, tb_ref, out_ref):
    x = pooled_ref[...] / den_ref[...]
    x = (
        jnp.dot(wt_ref[...], x, preferred_element_type=jnp.float32)
        + tb_ref[...]
    )
    v = v_ref[...]
    vn = v / jnp.maximum(
        jnp.sqrt(jnp.sum(v * v, axis=0, keepdims=True)), 1e-12
    )
    xn = x / jnp.maximum(
        jnp.sqrt(jnp.sum(x * x, axis=0, keepdims=True)), 1e-12
    )
    cos = jnp.sum(vn * xn, axis=0, keepdims=True) / jnp.maximum(
        jnp.sqrt(jnp.sum(vn * vn, axis=0, keepdims=True))
        * jnp.sqrt(jnp.sum(xn * xn, axis=0, keepdims=True)),
        1e-8,
    )
    loss = jnp.sum(1.0 - cos) * (1.0 / B)
    out_ref[...] = jnp.reshape(loss, (1, 1))


def _tc_final(pooled_t, v_t, den_t, txt_w, txt_b2):
    out = pl.pallas_call(
        _final_body,
        grid=(1,),
        in_specs=[
            pl.BlockSpec((D, B), lambda i: (0, 0)),
            pl.BlockSpec((D, B), lambda i: (0, 0)),
            pl.BlockSpec((1, B), lambda i: (0, 0)),
            pl.BlockSpec((D, D), lambda i: (0, 0)),
            pl.BlockSpec((D, 1), lambda i: (0, 0)),
        ],
        out_specs=pl.BlockSpec((1, 1), lambda i: (0, 0)),
        out_shape=jax.ShapeDtypeStruct((1, 1), jnp.float32),
    )(pooled_t, v_t, den_t, txt_w, txt_b2)
    return out[0, 0]


def kernel(video, caption_ids, emb_table, txt_w, txt_b, vid_w, vid_b):
    ids = caption_ids.astype(jnp.int32)
    pooled = _sc_pool_sums(emb_table, ids.reshape(B * 2, HALF))
    vid2 = video.transpose(1, 2, 3, 4, 0).reshape(12 * 256, B)
    ids_t = ids.T
    w_big = jnp.tile(jnp.repeat(vid_w * (1.0 / 1024.0), 256, axis=1), (1, 4))
    v_t, den_t = _tc_video(vid2, ids_t, w_big, vid_b.reshape(D, 1))
    return _tc_final(pooled.T, v_t, den_t, txt_w, txt_b.reshape(D, 1))
